# Initial kernel scaffold; baseline (speedup 1.0000x reference)
import jax
import jax.numpy as jnp
from jax.experimental import pallas as pl


def kernel(users, items_i, items_j, visfeat_i, visfeat_j, text_i, text_j, user_alpha, item_alpha, user_beta, item_beta, user_visembed, user_textembed, vis_W, vis_b, text_table, conv_W2, conv_b2, conv_W3, conv_b3, conv_W4, conv_b4, conv_W5, conv_b5, textnn_W, textnn_b):
    raise NotImplementedError("write your pallas kernel here")


# speedup vs baseline: 3.3726x; 3.3726x over previous
import functools

import jax
import jax.numpy as jnp
from jax import lax
from jax.experimental import pallas as pl
from jax.experimental.pallas import tpu as pltpu
from jax.experimental.pallas import tpu_sc as plsc

U = 100000
NI = 100000
V = 100000
D = 512
VIS = 2048
L = 83
T = 300
B = 1024

KS = (2, 3, 4, 5)
NG = sum(KS)
GW = 128
TP = 384


_NW = 32
_ROWS_PW = (B * L) // _NW
_CH = 128
_NFULL = _ROWS_PW // _CH
_TAIL_OFF = _ROWS_PW - _CH
_BPW = B // _NW


def _sc_gather_body(ti, tj, users, items_i, items_j, bri, brj,
                    text_table, user_alpha, user_vis, user_txt,
                    item_alpha, beta_tbl,
                    emb_i, emb_j, ua, tv, tt, ia_i, ia_j, ib_i, ib_j,
                    idx_v, rows_v, idxb_v, rows512_v, rowsbeta_v, sem):
    nc = plsc.get_sparse_core_info().num_cores
    wid = lax.axis_index("s") * nc + lax.axis_index("c")
    tbase = pl.multiple_of(wid * _ROWS_PW, 8)

    def gather_side(src_idx, dst):
        def chunk(c, _):
            off = pl.multiple_of(tbase + c * _CH, 8)
            pltpu.sync_copy(src_idx.at[pl.ds(off, _CH)], idx_v)
            pltpu.async_copy(text_table.at[idx_v], rows_v, sem).wait()
            pltpu.sync_copy(rows_v, dst.at[pl.ds(off, _CH)])
            return ()
        lax.fori_loop(0, _NFULL, chunk, ())
        off = pl.multiple_of(tbase + _TAIL_OFF, 8)
        pltpu.sync_copy(src_idx.at[pl.ds(off, _CH)], idx_v)
        pltpu.async_copy(text_table.at[idx_v], rows_v, sem).wait()
        pltpu.sync_copy(rows_v, dst.at[pl.ds(off, _CH)])

    gather_side(ti, emb_i)
    gather_side(tj, emb_j)

    bbase = pl.multiple_of(wid * _BPW, 8)

    def gather_rows(idx_src, table, dst):
        pltpu.sync_copy(idx_src.at[pl.ds(bbase, _BPW)], idxb_v)
        pltpu.async_copy(table.at[idxb_v], rows512_v, sem).wait()
        pltpu.sync_copy(rows512_v, dst.at[pl.ds(bbase, _BPW)])

    gather_rows(users, user_alpha, ua)
    gather_rows(users, user_vis, tv)
    gather_rows(users, user_txt, tt)
    gather_rows(items_i, item_alpha, ia_i)
    gather_rows(items_j, item_alpha, ia_j)

    def gather_beta(idx_src, dst):
        pltpu.sync_copy(idx_src.at[pl.ds(bbase, _BPW)], idxb_v)
        pltpu.async_copy(beta_tbl.at[idxb_v], rowsbeta_v, sem).wait()
        pltpu.sync_copy(rowsbeta_v, dst.at[pl.ds(bbase, _BPW)])

    gather_beta(bri, ib_i)
    gather_beta(brj, ib_j)


def _sc_gather(ti, tj, users, items_i, items_j, bri, brj,
               text_table, user_alpha, user_vis, user_txt,
               item_alpha, beta_tbl):
    f32 = jnp.float32
    out_type = (
        jax.ShapeDtypeStruct((B * L, TP), f32),
        jax.ShapeDtypeStruct((B * L, TP), f32),
        jax.ShapeDtypeStruct((B, D), f32),
        jax.ShapeDtypeStruct((B, D), f32),
        jax.ShapeDtypeStruct((B, D), f32),
        jax.ShapeDtypeStruct((B, D), f32),
        jax.ShapeDtypeStruct((B, D), f32),
        jax.ShapeDtypeStruct((B, 128), f32),
        jax.ShapeDtypeStruct((B, 128), f32),
    )
    kern = functools.partial(
        pl.kernel,
        mesh=plsc.VectorSubcoreMesh(core_axis_name="c", subcore_axis_name="s"),
        out_type=out_type,
        scratch_types=[
            pltpu.VMEM((_CH,), jnp.int32),
            pltpu.VMEM((_CH, TP), f32),
            pltpu.VMEM((_BPW,), jnp.int32),
            pltpu.VMEM((_BPW, D), f32),
            pltpu.VMEM((_BPW, 128), f32),
            pltpu.SemaphoreType.DMA,
        ],
    )(_sc_gather_body)
    return kern(ti, tj, users, items_i, items_j, bri, brj, text_table,
                user_alpha, user_vis, user_txt, item_alpha, beta_tbl)


_BB = 8
_GRID = B // _BB


def _tc_body(emb_i, emb_j, vf_i, vf_j, ua, ia_i, ia_j, tv, tt,
             ibr_i, ibr_j, ci, cj, wcat, bcat, tw, tb, vw, vb, out):
    f32 = jnp.float32

    def txt_branch(emb_ref):
        emb = emb_ref[...]
        feats = []
        gbase = 0
        for k in KS:
            w = wcat[:, gbase * GW:(gbase + k) * GW]
            z = lax.dot_general(emb, w, (((1,), (0,)), ((), ())),
                                preferred_element_type=f32)
            z3 = z.reshape(_BB, L, k * GW)
            n = L - k + 1
            acc = z3[:, 0:n, 0:GW]
            for dk in range(1, k):
                acc = acc + z3[:, dk:dk + n, dk * GW:(dk + 1) * GW]
            feats.append(jnp.max(acc, axis=1))
            gbase += k
        m = jnp.concatenate(feats, axis=-1)
        h = jax.nn.sigmoid(m + bcat[...])
        return jax.nn.sigmoid(
            lax.dot_general(h, tw[...], (((1,), (0,)), ((), ())),
                            preferred_element_type=f32) + tb[...])

    txt_i = txt_branch(emb_i)
    txt_j = txt_branch(emb_j)

    def vis_branch(vf_ref):
        return jax.nn.sigmoid(
            lax.dot_general(vf_ref[...], vw[...], (((1,), (0,)), ((), ())),
                            preferred_element_type=f32) + vb[...])

    vis_i = vis_branch(vf_i)
    vis_j = vis_branch(vf_j)

    lane = lax.broadcasted_iota(jnp.int32, (_BB, 128), 1)
    bi = jnp.sum(jnp.where(lane == ci[...], ibr_i[...], 0.0),
                 axis=-1, keepdims=True)
    bj = jnp.sum(jnp.where(lane == cj[...], ibr_j[...], 0.0),
                 axis=-1, keepdims=True)
    s = bi - bj
    s = s + jnp.sum(ua[...] * (ia_i[...] - ia_j[...]), axis=-1, keepdims=True)
    s = s + jnp.sum(tv[...] * (vis_i - vis_j), axis=-1, keepdims=True)
    s = s + jnp.sum(tt[...] * (txt_i - txt_j), axis=-1, keepdims=True)
    out[...] = s


def _tc_compute(emb_i, emb_j, vf_i, vf_j, ua, ia_i, ia_j, tv, tt,
                ibr_i, ibr_j, ci, cj, wcat, bcat, tw, tb, vw, vb):
    f32 = jnp.float32
    row_blk = lambda r, c: pl.BlockSpec((r, c), lambda i: (i, 0))
    full_blk = lambda r, c: pl.BlockSpec((r, c), lambda i: (0, 0))
    return pl.pallas_call(
        _tc_body,
        grid=(_GRID,),
        in_specs=[
            row_blk(_BB * L, TP),
            row_blk(_BB * L, TP),
            row_blk(_BB, VIS),
            row_blk(_BB, VIS),
            row_blk(_BB, D),
            row_blk(_BB, D),
            row_blk(_BB, D),
            row_blk(_BB, D),
            row_blk(_BB, D),
            row_blk(_BB, 128),
            row_blk(_BB, 128),
            row_blk(_BB, 1),
            row_blk(_BB, 1),
            full_blk(TP, NG * GW),
            full_blk(1, 4 * GW),
            full_blk(4 * GW, D),
            full_blk(1, D),
            full_blk(VIS, D),
            full_blk(1, D),
        ],
        out_specs=row_blk(_BB, 1),
        out_shape=jax.ShapeDtypeStruct((B, 1), f32),
    )(emb_i, emb_j, vf_i, vf_j, ua, ia_i, ia_j, tv, tt, ibr_i, ibr_j, ci, cj,
      wcat, bcat, tw, tb, vw, vb)


def _prep_weights(conv_Ws, conv_bs, textnn_W, textnn_b, vis_b):
    f32 = jnp.float32
    wcat = jnp.zeros((TP, NG * GW), f32)
    g = 0
    for k, wk in zip(KS, conv_Ws):
        for dk in range(k):
            wcat = wcat.at[0:T, g * GW:g * GW + 100].set(wk[:, 0, dk, :].T)
            g += 1
    bcat = jnp.zeros((1, 4 * GW), f32)
    tw = jnp.zeros((4 * GW, D), f32)
    for c, bk in enumerate(conv_bs):
        bcat = bcat.at[0, c * GW:c * GW + 100].set(bk)
        tw = tw.at[c * GW:c * GW + 100, :].set(textnn_W[c * 100:(c + 1) * 100, :])
    return wcat, bcat, tw, textnn_b.reshape(1, D), vis_b.reshape(1, D)


def kernel(users, items_i, items_j, visfeat_i, visfeat_j, text_i, text_j,
           user_alpha, item_alpha, user_beta, item_beta, user_visembed,
           user_textembed, vis_W, vis_b, text_table, conv_W2, conv_b2,
           conv_W3, conv_b3, conv_W4, conv_b4, conv_W5, conv_b5,
           textnn_W, textnn_b):
    del user_beta
    i32 = jnp.int32
    ti = text_i.reshape(-1).astype(i32)
    tj = text_j.reshape(-1).astype(i32)
    u = users.astype(i32)
    ii = items_i.astype(i32)
    ij = items_j.astype(i32)

    table_p = jnp.pad(text_table, ((0, 0), (0, TP - T)))
    beta_tbl = jnp.pad(item_beta.reshape(-1), (0, 782 * 128 - NI)).reshape(782, 128)
    bri = jax.lax.shift_right_logical(ii, 7)
    brj = jax.lax.shift_right_logical(ij, 7)
    ci = jax.lax.bitwise_and(ii, 127).reshape(B, 1)
    cj = jax.lax.bitwise_and(ij, 127).reshape(B, 1)
    (emb_i, emb_j, ua, tv, tt, ia_i, ia_j, ibr_i, ibr_j) = _sc_gather(
        ti, tj, u, ii, ij, bri, brj, table_p, user_alpha, user_visembed,
        user_textembed, item_alpha, beta_tbl)

    wcat, bcat, tw, tb, vb = _prep_weights(
        (conv_W2, conv_W3, conv_W4, conv_W5),
        (conv_b2, conv_b3, conv_b4, conv_b5),
        textnn_W, textnn_b, vis_b)

    out = _tc_compute(emb_i, emb_j, visfeat_i, visfeat_j, ua, ia_i, ia_j,
                      tv, tt, ibr_i, ibr_j, ci, cj, wcat, bcat, tw, tb,
                      vis_W, vb)
    return out.reshape(B)


# speedup vs baseline: 4.1905x; 1.2425x over previous
import functools

import jax
import jax.numpy as jnp
from jax import lax
from jax.experimental import pallas as pl
from jax.experimental.pallas import tpu as pltpu
from jax.experimental.pallas import tpu_sc as plsc

U = 100000
NI = 100000
V = 100000
D = 512
VIS = 2048
L = 83
T = 300
B = 1024

KS = (2, 3, 4, 5)
NG = sum(KS)
GW = 128
TP = 384


_NW = 32
_ROWS_PW = (B * L) // _NW
_CH = 128
_NFULL = _ROWS_PW // _CH
_TAIL_OFF = _ROWS_PW - _CH
_BPW = B // _NW


def _sc_gather_body(ti, tj, users, items_i, items_j, bri, brj,
                    text_table, user_alpha, user_vis, user_txt,
                    item_alpha, beta_tbl,
                    emb_i, emb_j, ua, tv, tt, ia_i, ia_j, ib_i, ib_j,
                    idx_v, rows_v, idxb_v, rows512_v, rowsbeta_v, sem):
    nc = plsc.get_sparse_core_info().num_cores
    wid = lax.axis_index("s") * nc + lax.axis_index("c")
    tbase = pl.multiple_of(wid * _ROWS_PW, 8)

    def gather_side(src_idx, dst):
        def chunk(c, _):
            off = pl.multiple_of(tbase + c * _CH, 8)
            pltpu.sync_copy(src_idx.at[pl.ds(off, _CH)], idx_v)
            pltpu.async_copy(text_table.at[idx_v], rows_v, sem).wait()
            pltpu.sync_copy(rows_v, dst.at[pl.ds(off, _CH)])
            return ()
        lax.fori_loop(0, _NFULL, chunk, ())
        off = pl.multiple_of(tbase + _TAIL_OFF, 8)
        pltpu.sync_copy(src_idx.at[pl.ds(off, _CH)], idx_v)
        pltpu.async_copy(text_table.at[idx_v], rows_v, sem).wait()
        pltpu.sync_copy(rows_v, dst.at[pl.ds(off, _CH)])

    gather_side(ti, emb_i)
    gather_side(tj, emb_j)

    bbase = pl.multiple_of(wid * _BPW, 8)

    def gather_rows(idx_src, table, dst):
        pltpu.sync_copy(idx_src.at[pl.ds(bbase, _BPW)], idxb_v)
        pltpu.async_copy(table.at[idxb_v], rows512_v, sem).wait()
        pltpu.sync_copy(rows512_v, dst.at[pl.ds(bbase, _BPW)])

    gather_rows(users, user_alpha, ua)
    gather_rows(users, user_vis, tv)
    gather_rows(users, user_txt, tt)
    gather_rows(items_i, item_alpha, ia_i)
    gather_rows(items_j, item_alpha, ia_j)

    def gather_beta(idx_src, dst):
        pltpu.sync_copy(idx_src.at[pl.ds(bbase, _BPW)], idxb_v)
        pltpu.async_copy(beta_tbl.at[idxb_v], rowsbeta_v, sem).wait()
        pltpu.sync_copy(rowsbeta_v, dst.at[pl.ds(bbase, _BPW)])

    gather_beta(bri, ib_i)
    gather_beta(brj, ib_j)


def _sc_gather(ti, tj, users, items_i, items_j, bri, brj,
               text_table, user_alpha, user_vis, user_txt,
               item_alpha, beta_tbl):
    f32 = jnp.float32
    out_type = (
        jax.ShapeDtypeStruct((B * L, TP), f32),
        jax.ShapeDtypeStruct((B * L, TP), f32),
        jax.ShapeDtypeStruct((B, D), f32),
        jax.ShapeDtypeStruct((B, D), f32),
        jax.ShapeDtypeStruct((B, D), f32),
        jax.ShapeDtypeStruct((B, D), f32),
        jax.ShapeDtypeStruct((B, D), f32),
        jax.ShapeDtypeStruct((B, 128), f32),
        jax.ShapeDtypeStruct((B, 128), f32),
    )
    kern = functools.partial(
        pl.kernel,
        mesh=plsc.VectorSubcoreMesh(core_axis_name="c", subcore_axis_name="s"),
        out_type=out_type,
        scratch_types=[
            pltpu.VMEM((_CH,), jnp.int32),
            pltpu.VMEM((_CH, TP), f32),
            pltpu.VMEM((_BPW,), jnp.int32),
            pltpu.VMEM((_BPW, D), f32),
            pltpu.VMEM((_BPW, 128), f32),
            pltpu.SemaphoreType.DMA,
        ],
    )(_sc_gather_body)
    return kern(ti, tj, users, items_i, items_j, bri, brj, text_table,
                user_alpha, user_vis, user_txt, item_alpha, beta_tbl)


_VB = 1000


def _pad_body(src, dst):
    x = src[...]
    dst[...] = jnp.concatenate(
        [x, jnp.zeros((_VB, TP - T), jnp.float32)], axis=1)


def _pad_table(table):
    return pl.pallas_call(
        _pad_body,
        grid=(V // _VB,),
        in_specs=[pl.BlockSpec((_VB, T), lambda i: (i, 0))],
        out_specs=pl.BlockSpec((_VB, TP), lambda i: (i, 0)),
        out_shape=jax.ShapeDtypeStruct((V, TP), jnp.float32),
    )(table)


_BB = 8
_GRID = B // _BB


def _tc_body(emb_i, emb_j, vf_i, vf_j, ua, ia_i, ia_j, tv, tt,
             ibr_i, ibr_j, ci, cj, wcat, bcat, tw, tb, vw, vb, out):
    f32 = jnp.float32

    def txt_branch(emb_ref):
        emb = emb_ref[...].astype(jnp.bfloat16)
        feats = []
        gbase = 0
        for k in KS:
            w = wcat[:, gbase * GW:(gbase + k) * GW]
            z = lax.dot_general(emb, w, (((1,), (0,)), ((), ())),
                                preferred_element_type=f32)
            z3 = z.reshape(_BB, L, k * GW)
            n = L - k + 1
            acc = z3[:, 0:n, 0:GW]
            for dk in range(1, k):
                acc = acc + z3[:, dk:dk + n, dk * GW:(dk + 1) * GW]
            feats.append(jnp.max(acc, axis=1))
            gbase += k
        m = jnp.concatenate(feats, axis=-1)
        h = jax.nn.sigmoid(m + bcat[...])
        return jax.nn.sigmoid(
            lax.dot_general(h, tw[...], (((1,), (0,)), ((), ())),
                            preferred_element_type=f32) + tb[...])

    txt_i = txt_branch(emb_i)
    txt_j = txt_branch(emb_j)

    def vis_branch(vf_ref):
        return jax.nn.sigmoid(
            lax.dot_general(vf_ref[...], vw[...], (((1,), (0,)), ((), ())),
                            preferred_element_type=f32) + vb[...])

    vis_i = vis_branch(vf_i)
    vis_j = vis_branch(vf_j)

    lane = lax.broadcasted_iota(jnp.int32, (_BB, 128), 1)
    bi = jnp.sum(jnp.where(lane == ci[...], ibr_i[...], 0.0),
                 axis=-1, keepdims=True)
    bj = jnp.sum(jnp.where(lane == cj[...], ibr_j[...], 0.0),
                 axis=-1, keepdims=True)
    s = bi - bj
    s = s + jnp.sum(ua[...] * (ia_i[...] - ia_j[...]), axis=-1, keepdims=True)
    s = s + jnp.sum(tv[...] * (vis_i - vis_j), axis=-1, keepdims=True)
    s = s + jnp.sum(tt[...] * (txt_i - txt_j), axis=-1, keepdims=True)
    out[...] = s


def _tc_compute(emb_i, emb_j, vf_i, vf_j, ua, ia_i, ia_j, tv, tt,
                ibr_i, ibr_j, ci, cj, wcat, bcat, tw, tb, vw, vb):
    f32 = jnp.float32
    row_blk = lambda r, c: pl.BlockSpec((r, c), lambda i: (i, 0))
    full_blk = lambda r, c: pl.BlockSpec((r, c), lambda i: (0, 0))
    return pl.pallas_call(
        _tc_body,
        grid=(_GRID,),
        in_specs=[
            row_blk(_BB * L, TP),
            row_blk(_BB * L, TP),
            row_blk(_BB, VIS),
            row_blk(_BB, VIS),
            row_blk(_BB, D),
            row_blk(_BB, D),
            row_blk(_BB, D),
            row_blk(_BB, D),
            row_blk(_BB, D),
            row_blk(_BB, 128),
            row_blk(_BB, 128),
            row_blk(_BB, 1),
            row_blk(_BB, 1),
            full_blk(TP, NG * GW),
            full_blk(1, 4 * GW),
            full_blk(4 * GW, D),
            full_blk(1, D),
            full_blk(VIS, D),
            full_blk(1, D),
        ],
        out_specs=row_blk(_BB, 1),
        out_shape=jax.ShapeDtypeStruct((B, 1), f32),
    )(emb_i, emb_j, vf_i, vf_j, ua, ia_i, ia_j, tv, tt, ibr_i, ibr_j, ci, cj,
      wcat, bcat, tw, tb, vw, vb)


def _prep_weights(conv_Ws, conv_bs, textnn_W, textnn_b, vis_b):
    f32 = jnp.float32
    wcat = jnp.zeros((TP, NG * GW), f32)
    g = 0
    for k, wk in zip(KS, conv_Ws):
        for dk in range(k):
            wcat = wcat.at[0:T, g * GW:g * GW + 100].set(wk[:, 0, dk, :].T)
            g += 1
    wcat = wcat.astype(jnp.bfloat16)
    bcat = jnp.zeros((1, 4 * GW), f32)
    tw = jnp.zeros((4 * GW, D), f32)
    for c, bk in enumerate(conv_bs):
        bcat = bcat.at[0, c * GW:c * GW + 100].set(bk)
        tw = tw.at[c * GW:c * GW + 100, :].set(textnn_W[c * 100:(c + 1) * 100, :])
    return wcat, bcat, tw, textnn_b.reshape(1, D), vis_b.reshape(1, D)


def kernel(users, items_i, items_j, visfeat_i, visfeat_j, text_i, text_j,
           user_alpha, item_alpha, user_beta, item_beta, user_visembed,
           user_textembed, vis_W, vis_b, text_table, conv_W2, conv_b2,
           conv_W3, conv_b3, conv_W4, conv_b4, conv_W5, conv_b5,
           textnn_W, textnn_b):
    del user_beta
    i32 = jnp.int32
    ti = text_i.reshape(-1).astype(i32)
    tj = text_j.reshape(-1).astype(i32)
    u = users.astype(i32)
    ii = items_i.astype(i32)
    ij = items_j.astype(i32)

    table_p = _pad_table(text_table)
    beta_tbl = jnp.pad(item_beta.reshape(-1), (0, 782 * 128 - NI)).reshape(782, 128)
    bri = jax.lax.shift_right_logical(ii, 7)
    brj = jax.lax.shift_right_logical(ij, 7)
    ci = jax.lax.bitwise_and(ii, 127).reshape(B, 1)
    cj = jax.lax.bitwise_and(ij, 127).reshape(B, 1)
    (emb_i, emb_j, ua, tv, tt, ia_i, ia_j, ibr_i, ibr_j) = _sc_gather(
        ti, tj, u, ii, ij, bri, brj, table_p, user_alpha, user_visembed,
        user_textembed, item_alpha, beta_tbl)

    wcat, bcat, tw, tb, vb = _prep_weights(
        (conv_W2, conv_W3, conv_W4, conv_W5),
        (conv_b2, conv_b3, conv_b4, conv_b5),
        textnn_W, textnn_b, vis_b)

    out = _tc_compute(emb_i, emb_j, visfeat_i, visfeat_j, ua, ia_i, ia_j,
                      tv, tt, ibr_i, ibr_j, ci, cj, wcat, bcat, tw, tb,
                      vis_W, vb)
    return out.reshape(B)
